# baseline (speedup 1.0000x reference)
import jax
import jax.numpy as jnp
from jax import lax
from jax.experimental import pallas as pl
from jax.experimental.pallas import tpu as pltpu
from jax.experimental.pallas import tpu_sc as plsc

NODE_SIZE = 100000
EMBED_SIZE = 128
BATCH = 16384

L = 16
NW = 32
B_PER_W = BATCH // NW
CHUNK = 128
CHUNKS = (64, 64, 128, 128, 128)
NBUF = 3
UNROLL = 4

_LOG1P_COEF = (
    1.446112683e-03, -1.027147447e-02, 3.423174471e-02, -7.301764925e-02,
    1.166124657e-01, -1.571737904e-01, 1.976391457e-01, -2.496172750e-01,
    3.332960370e-01, -4.999980978e-01, 9.999999616e-01, 0.0,
)


def _log1p_poly(u):
    acc = jnp.full((L,), _LOG1P_COEF[0], dtype=jnp.float32)
    for c in _LOG1P_COEF[1:]:
        acc = acc * u + c
    return acc


def _sc_kernel(a_hbm, b_hbm, sign_hbm, emb_hbm, ctx_hbm, out_hbm,
               idx_a, idx_b, sign_v, dots, rows, sems, sem_i, sem_s):
    wid = lax.axis_index("s") * 2 + lax.axis_index("c")
    base = wid * B_PER_W

    cp_ia = pltpu.async_copy(a_hbm.at[pl.ds(base, B_PER_W)], idx_a, sem_i)
    cp_ib = pltpu.async_copy(b_hbm.at[pl.ds(base, B_PER_W)], idx_b, sem_i)
    cp_sg = pltpu.async_copy(sign_hbm.at[pl.ds(base, B_PER_W)], sign_v, sem_s)
    cp_ia.wait()
    cp_ib.wait()
    cp_sg.wait()

    lane = lax.iota(jnp.int32, L)
    perms = {h: lane ^ h for h in (8, 4, 2, 1)}
    mask0 = lane == 0

    offs = []
    o = 0
    for n in CHUNKS:
        offs.append(o)
        o += n

    def start(c):
        s = c % NBUF
        n, off = CHUNKS[c], offs[c]
        cp_a = pltpu.async_copy(
            emb_hbm.at[idx_a.at[pl.ds(off, n)]],
            rows[2 * s].at[pl.ds(0, n)], sems[2 * s])
        cp_b = pltpu.async_copy(
            ctx_hbm.at[idx_b.at[pl.ds(off, n)]],
            rows[2 * s + 1].at[pl.ds(0, n)], sems[2 * s + 1])
        return cp_a, cp_b

    inflight = [start(c) for c in range(NBUF - 1)]
    for c in range(len(CHUNKS)):
        s = c % NBUF
        arows, brows = rows[2 * s], rows[2 * s + 1]
        cp_a, cp_b = inflight.pop(0)
        cp_a.wait()
        cp_b.wait()
        if c + NBUF - 1 < len(CHUNKS):
            inflight.append(start(c + NBUF - 1))

        @plsc.parallel_loop(0, CHUNKS[c], unroll=UNROLL)
        def _dot_body(r, c=c, arows=arows, brows=brows):
            acc0 = (arows[r, pl.ds(0, L)] * brows[r, pl.ds(0, L)])
            acc1 = (arows[r, pl.ds(L, L)] * brows[r, pl.ds(L, L)])
            for j in range(2, EMBED_SIZE // L, 2):
                acc0 = acc0 + arows[r, pl.ds(j * L, L)] * brows[r, pl.ds(j * L, L)]
                acc1 = acc1 + arows[r, pl.ds((j + 1) * L, L)] * brows[r, pl.ds((j + 1) * L, L)]
            v = acc0 + acc1
            for h in (8, 4, 2, 1):
                v = v + jnp.take(v, perms[h])
            plsc.store_scatter(dots, [jnp.full((L,), offs[c] + r, jnp.int32)],
                               v, mask=mask0)

    @plsc.parallel_loop(0, B_PER_W // L, unroll=4)
    def _loss_body(i):
        off = pl.multiple_of(i * L, L)
        t = sign_v[pl.ds(off, L)] * dots[pl.ds(off, L)]
        u = jnp.exp(-jnp.abs(t))
        dots[pl.ds(off, L)] = jnp.maximum(-t, 0.0) + _log1p_poly(u)

    pltpu.sync_copy(dots, out_hbm.at[pl.ds(base, B_PER_W)])


@jax.jit
def _run(a1, b1, sign, emb_table, ctx_table):
    mesh = plsc.VectorSubcoreMesh(core_axis_name="c", subcore_axis_name="s")
    f = pl.kernel(
        _sc_kernel,
        mesh=mesh,
        compiler_params=pltpu.CompilerParams(needs_layout_passes=False),
        out_type=jax.ShapeDtypeStruct((BATCH,), jnp.float32),
        scratch_types=[
            pltpu.VMEM((B_PER_W,), jnp.int32),
            pltpu.VMEM((B_PER_W,), jnp.int32),
            pltpu.VMEM((B_PER_W,), jnp.float32),
            pltpu.VMEM((B_PER_W,), jnp.float32),
            [pltpu.VMEM((CHUNK, EMBED_SIZE), jnp.float32)
             for _ in range(2 * NBUF)],
            [pltpu.SemaphoreType.DMA for _ in range(2 * NBUF)],
            pltpu.SemaphoreType.DMA,
            pltpu.SemaphoreType.DMA,
        ],
    )
    return f(a1, b1, sign, emb_table, ctx_table)


def kernel(a, b, sign, emb_table, ctx_table):
    return _run(a, b, sign, emb_table, ctx_table)


# speedup vs baseline: 1.0120x; 1.0120x over previous
import jax
import jax.numpy as jnp
from jax import lax
from jax.experimental import pallas as pl
from jax.experimental.pallas import tpu as pltpu
from jax.experimental.pallas import tpu_sc as plsc

NODE_SIZE = 100000
EMBED_SIZE = 128
BATCH = 16384

L = 16
NW = 32
B_PER_W = BATCH // NW
CHUNK = 128
CHUNKS = (128, 128, 128, 128)
NBUF = 3
UNROLL = 4

_LOG1P_COEF = (
    1.446112683e-03, -1.027147447e-02, 3.423174471e-02, -7.301764925e-02,
    1.166124657e-01, -1.571737904e-01, 1.976391457e-01, -2.496172750e-01,
    3.332960370e-01, -4.999980978e-01, 9.999999616e-01, 0.0,
)


def _log1p_poly(u):
    acc = jnp.full((L,), _LOG1P_COEF[0], dtype=jnp.float32)
    for c in _LOG1P_COEF[1:]:
        acc = acc * u + c
    return acc


def _sc_kernel(a_hbm, b_hbm, sign_hbm, emb_hbm, ctx_hbm, out_hbm,
               idx_a, idx_b, sign_v, dots, rows, sems, sem_i, sem_s):
    wid = lax.axis_index("s") * 2 + lax.axis_index("c")
    base = wid * B_PER_W

    cp_ia = pltpu.async_copy(a_hbm.at[pl.ds(base, B_PER_W)], idx_a, sem_i)
    cp_ib = pltpu.async_copy(b_hbm.at[pl.ds(base, B_PER_W)], idx_b, sem_i)
    cp_sg = pltpu.async_copy(sign_hbm.at[pl.ds(base, B_PER_W)], sign_v, sem_s)
    cp_ia.wait()
    cp_ib.wait()
    cp_sg.wait()

    lane = lax.iota(jnp.int32, L)
    perms = {h: lane ^ h for h in (8, 4, 2, 1)}
    mask0 = lane == 0

    offs = []
    o = 0
    for n in CHUNKS:
        offs.append(o)
        o += n

    def start(c):
        s = c % NBUF
        n, off = CHUNKS[c], offs[c]
        cp_a = pltpu.async_copy(
            emb_hbm.at[idx_a.at[pl.ds(off, n)]],
            rows[2 * s].at[pl.ds(0, n)], sems[2 * s])
        cp_b = pltpu.async_copy(
            ctx_hbm.at[idx_b.at[pl.ds(off, n)]],
            rows[2 * s + 1].at[pl.ds(0, n)], sems[2 * s + 1])
        return cp_a, cp_b

    inflight = [start(c) for c in range(NBUF - 1)]
    for c in range(len(CHUNKS)):
        s = c % NBUF
        arows, brows = rows[2 * s], rows[2 * s + 1]
        cp_a, cp_b = inflight.pop(0)
        cp_a.wait()
        cp_b.wait()
        if c + NBUF - 1 < len(CHUNKS):
            inflight.append(start(c + NBUF - 1))

        @plsc.parallel_loop(0, CHUNKS[c], unroll=UNROLL)
        def _dot_body(r, c=c, arows=arows, brows=brows):
            acc0 = (arows[r, pl.ds(0, L)] * brows[r, pl.ds(0, L)])
            acc1 = (arows[r, pl.ds(L, L)] * brows[r, pl.ds(L, L)])
            for j in range(2, EMBED_SIZE // L, 2):
                acc0 = acc0 + arows[r, pl.ds(j * L, L)] * brows[r, pl.ds(j * L, L)]
                acc1 = acc1 + arows[r, pl.ds((j + 1) * L, L)] * brows[r, pl.ds((j + 1) * L, L)]
            v = acc0 + acc1
            for h in (8, 4, 2, 1):
                v = v + jnp.take(v, perms[h])
            plsc.store_scatter(dots, [jnp.full((L,), offs[c] + r, jnp.int32)],
                               v, mask=mask0)

    @plsc.parallel_loop(0, B_PER_W // L, unroll=4)
    def _loss_body(i):
        off = pl.multiple_of(i * L, L)
        t = sign_v[pl.ds(off, L)] * dots[pl.ds(off, L)]
        u = jnp.exp(-jnp.abs(t))
        dots[pl.ds(off, L)] = jnp.maximum(-t, 0.0) + _log1p_poly(u)

    pltpu.sync_copy(dots, out_hbm.at[pl.ds(base, B_PER_W)])


@jax.jit
def _run(a1, b1, sign, emb_table, ctx_table):
    mesh = plsc.VectorSubcoreMesh(core_axis_name="c", subcore_axis_name="s")
    f = pl.kernel(
        _sc_kernel,
        mesh=mesh,
        compiler_params=pltpu.CompilerParams(needs_layout_passes=False),
        out_type=jax.ShapeDtypeStruct((BATCH,), jnp.float32),
        scratch_types=[
            pltpu.VMEM((B_PER_W,), jnp.int32),
            pltpu.VMEM((B_PER_W,), jnp.int32),
            pltpu.VMEM((B_PER_W,), jnp.float32),
            pltpu.VMEM((B_PER_W,), jnp.float32),
            [pltpu.VMEM((CHUNK, EMBED_SIZE), jnp.float32)
             for _ in range(2 * NBUF)],
            [pltpu.SemaphoreType.DMA for _ in range(2 * NBUF)],
            pltpu.SemaphoreType.DMA,
            pltpu.SemaphoreType.DMA,
        ],
    )
    return f(a1, b1, sign, emb_table, ctx_table)


def kernel(a, b, sign, emb_table, ctx_table):
    return _run(a, b, sign, emb_table, ctx_table)


# speedup vs baseline: 1.0124x; 1.0004x over previous
import jax
import jax.numpy as jnp
from jax import lax
from jax.experimental import pallas as pl
from jax.experimental.pallas import tpu as pltpu
from jax.experimental.pallas import tpu_sc as plsc

NODE_SIZE = 100000
EMBED_SIZE = 128
BATCH = 16384

L = 16
NW = 32
B_PER_W = BATCH // NW
CHUNK = 128
CHUNKS = (128, 128, 128, 128)
NBUF = 3
UNROLL = 4

_LOG1P_COEF = (
    1.446112683e-03, -1.027147447e-02, 3.423174471e-02, -7.301764925e-02,
    1.166124657e-01, -1.571737904e-01, 1.976391457e-01, -2.496172750e-01,
    3.332960370e-01, -4.999980978e-01, 9.999999616e-01, 0.0,
)


def _log1p_poly(u):
    acc = jnp.full((L,), _LOG1P_COEF[0], dtype=jnp.float32)
    for c in _LOG1P_COEF[1:]:
        acc = acc * u + c
    return acc


def _sc_kernel(a_hbm, b_hbm, sign_hbm, emb_hbm, ctx_hbm, out_hbm,
               idx_a, idx_b, sign_v, dots, rows, sems, sem_i, sem_s):
    wid = lax.axis_index("s") * 2 + lax.axis_index("c")
    base = wid * B_PER_W

    cp_ia = pltpu.async_copy(a_hbm.at[pl.ds(base, B_PER_W)], idx_a, sem_i)
    cp_ib = pltpu.async_copy(b_hbm.at[pl.ds(base, B_PER_W)], idx_b, sem_i)
    cp_sg = pltpu.async_copy(sign_hbm.at[pl.ds(base, B_PER_W)], sign_v, sem_s)
    cp_ia.wait()
    cp_ib.wait()

    lane = lax.iota(jnp.int32, L)
    perms = {h: lane ^ h for h in (8, 4, 2, 1)}
    mask0 = lane == 0

    offs = []
    o = 0
    for n in CHUNKS:
        offs.append(o)
        o += n

    def start(c):
        s = c % NBUF
        n, off = CHUNKS[c], offs[c]
        cp_a = pltpu.async_copy(
            emb_hbm.at[idx_a.at[pl.ds(off, n)]],
            rows[2 * s].at[pl.ds(0, n)], sems[2 * s])
        cp_b = pltpu.async_copy(
            ctx_hbm.at[idx_b.at[pl.ds(off, n)]],
            rows[2 * s + 1].at[pl.ds(0, n)], sems[2 * s + 1])
        return cp_a, cp_b

    inflight = [start(c) for c in range(NBUF - 1)]
    for c in range(len(CHUNKS)):
        s = c % NBUF
        arows, brows = rows[2 * s], rows[2 * s + 1]
        cp_a, cp_b = inflight.pop(0)
        cp_a.wait()
        cp_b.wait()
        if c + NBUF - 1 < len(CHUNKS):
            inflight.append(start(c + NBUF - 1))

        @plsc.parallel_loop(0, CHUNKS[c], unroll=UNROLL)
        def _dot_body(r, c=c, arows=arows, brows=brows):
            acc0 = (arows[r, pl.ds(0, L)] * brows[r, pl.ds(0, L)])
            acc1 = (arows[r, pl.ds(L, L)] * brows[r, pl.ds(L, L)])
            for j in range(2, EMBED_SIZE // L, 2):
                acc0 = acc0 + arows[r, pl.ds(j * L, L)] * brows[r, pl.ds(j * L, L)]
                acc1 = acc1 + arows[r, pl.ds((j + 1) * L, L)] * brows[r, pl.ds((j + 1) * L, L)]
            v = acc0 + acc1
            for h in (8, 4, 2, 1):
                v = v + jnp.take(v, perms[h])
            plsc.store_scatter(dots, [jnp.full((L,), offs[c] + r, jnp.int32)],
                               v, mask=mask0)

    cp_sg.wait()

    @plsc.parallel_loop(0, B_PER_W // L, unroll=4)
    def _loss_body(i):
        off = pl.multiple_of(i * L, L)
        t = sign_v[pl.ds(off, L)] * dots[pl.ds(off, L)]
        u = jnp.exp(-jnp.abs(t))
        dots[pl.ds(off, L)] = jnp.maximum(-t, 0.0) + _log1p_poly(u)

    pltpu.sync_copy(dots, out_hbm.at[pl.ds(base, B_PER_W)])


@jax.jit
def _run(a1, b1, sign, emb_table, ctx_table):
    mesh = plsc.VectorSubcoreMesh(core_axis_name="c", subcore_axis_name="s")
    f = pl.kernel(
        _sc_kernel,
        mesh=mesh,
        compiler_params=pltpu.CompilerParams(needs_layout_passes=False),
        out_type=jax.ShapeDtypeStruct((BATCH,), jnp.float32),
        scratch_types=[
            pltpu.VMEM((B_PER_W,), jnp.int32),
            pltpu.VMEM((B_PER_W,), jnp.int32),
            pltpu.VMEM((B_PER_W,), jnp.float32),
            pltpu.VMEM((B_PER_W,), jnp.float32),
            [pltpu.VMEM((CHUNK, EMBED_SIZE), jnp.float32)
             for _ in range(2 * NBUF)],
            [pltpu.SemaphoreType.DMA for _ in range(2 * NBUF)],
            pltpu.SemaphoreType.DMA,
            pltpu.SemaphoreType.DMA,
        ],
    )
    return f(a1, b1, sign, emb_table, ctx_table)


def kernel(a, b, sign, emb_table, ctx_table):
    return _run(a, b, sign, emb_table, ctx_table)
